# baseline (speedup 1.0000x reference)
import functools

import jax
import jax.numpy as jnp
from jax.experimental import pallas as pl
from jax.experimental.pallas import tpu as pltpu

B = 32
LQ = 32
LK = 2048
D = 1024
N_HEAD = 16
D_H = 64
FILT = 10

_SROWS = 16
_SCOLS = LK // _SROWS


def _scores_topk_kernel(q0_ref, k_ref, sidx_ref, fk_ref):
    kb = k_ref[0].reshape(_SROWS, _SCOLS, D)
    q0 = q0_ref[0].reshape(1, 1, D)
    q0 = q0.astype(jnp.bfloat16).astype(jnp.float32)

    acc = jnp.zeros((_SROWS, _SCOLS), jnp.float32)
    for c in range(0, D, 128):
        kc = kb[:, :, c:c + 128].astype(jnp.bfloat16).astype(jnp.float32)
        acc = acc + jnp.sum(kc * q0[:, :, c:c + 128], axis=-1)

    lin = (jax.lax.broadcasted_iota(jnp.int32, (_SROWS, _SCOLS), 0) * _SCOLS
           + jax.lax.broadcasted_iota(jnp.int32, (_SROWS, _SCOLS), 1))
    lane16 = jax.lax.broadcasted_iota(jnp.int32, (1, 16), 1)

    idx_vec = jnp.zeros((1, 16), jnp.int32)
    s = acc
    for j in range(FILT):
        m = jnp.max(s)
        idx = jnp.min(jnp.where(s >= m, lin, jnp.int32(LK)))
        s = jnp.where(lin == idx, -jnp.inf, s)
        idx_vec = jnp.where(lane16 == j, idx, idx_vec)
        fk_ref[0, j, :] = k_ref[0, pl.ds(idx, 1), :].reshape(D)
    sidx_ref[0] = idx_vec


def _gather_kernel(sidx_ref, v_ref, fv_ref):
    del sidx_ref
    fv_ref[...] = v_ref[...]


def _attn_kernel(q_ref, fk_ref, fv_ref, out_ref):
    qb = q_ref[0]
    kb = fk_ref[0]
    vb = fv_ref[0]
    for h in range(N_HEAD):
        sl = slice(h * D_H, (h + 1) * D_H)
        qh = qb[:, sl].astype(jnp.bfloat16)
        kh = kb[:, sl].astype(jnp.bfloat16)
        vh = vb[:, sl].astype(jnp.bfloat16)
        att = jax.lax.dot_general(qh, kh, (((1,), (1,)), ((), ())),
                                  preferred_element_type=jnp.float32)
        att = att - jnp.max(att, axis=1, keepdims=True)
        e = jnp.exp(att)
        p = e / jnp.sum(e, axis=1, keepdims=True)
        out_ref[0, :, sl] = jax.lax.dot_general(
            p.astype(jnp.bfloat16), vh, (((1,), (0,)), ((), ())),
            preferred_element_type=jnp.float32)


def kernel(q, k, v):
    q0 = q[:, 0:1, :]

    sidx3, f_k = pl.pallas_call(
        _scores_topk_kernel,
        grid=(B,),
        in_specs=[
            pl.BlockSpec((1, 1, D), lambda b: (b, 0, 0)),
            pl.BlockSpec((1, LK, D), lambda b: (b, 0, 0)),
        ],
        out_specs=[
            pl.BlockSpec((1, 1, 16), lambda b: (b, 0, 0)),
            pl.BlockSpec((1, FILT, D), lambda b: (b, 0, 0)),
        ],
        out_shape=[
            jax.ShapeDtypeStruct((B, 1, 16), jnp.int32),
            jax.ShapeDtypeStruct((B, FILT, D), jnp.float32),
        ],
    )(q0, k)

    s_index = sidx3[:, 0, :FILT]

    v4 = v.reshape(B, LK, 1, D)
    f_v = pl.pallas_call(
        _gather_kernel,
        grid_spec=pltpu.PrefetchScalarGridSpec(
            num_scalar_prefetch=1,
            grid=(B, FILT),
            in_specs=[
                pl.BlockSpec((1, 1, 1, D),
                             lambda b, j, sidx: (b, sidx[b, j], 0, 0)),
            ],
            out_specs=pl.BlockSpec((1, 1, 1, D),
                                   lambda b, j, sidx: (b, j, 0, 0)),
        ),
        out_shape=jax.ShapeDtypeStruct((B, FILT, 1, D), jnp.float32),
    )(s_index, v4)
    f_v = f_v.reshape(B, FILT, D)

    out = pl.pallas_call(
        _attn_kernel,
        grid=(B,),
        in_specs=[
            pl.BlockSpec((1, LQ, D), lambda b: (b, 0, 0)),
            pl.BlockSpec((1, FILT, D), lambda b: (b, 0, 0)),
            pl.BlockSpec((1, FILT, D), lambda b: (b, 0, 0)),
        ],
        out_specs=pl.BlockSpec((1, LQ, D), lambda b: (b, 0, 0)),
        out_shape=jax.ShapeDtypeStruct((B, LQ, D), jnp.float32),
    )(q, f_k, f_v)

    return out


# speedup vs baseline: 2.8416x; 2.8416x over previous
import jax
import jax.numpy as jnp
from jax.experimental import pallas as pl
from jax.experimental.pallas import tpu as pltpu

B = 32
LQ = 32
LK = 2048
D = 1024
N_HEAD = 16
D_H = 64
FILT = 10

_SROWS = 16
_SCOLS = LK // _SROWS


def _fused_kernel(q0t_ref, k_ref, q_ref, v_ref, out_ref, fk_ref, fv_ref, sem):
    b = pl.program_id(0)

    kb_bf = k_ref[0].astype(jnp.bfloat16)
    q0col = q0t_ref[0].astype(jnp.bfloat16)
    q0m = jnp.broadcast_to(q0col, (D, _SCOLS))
    s_full = jax.lax.dot_general(kb_bf, q0m, (((1,), (0,)), ((), ())),
                                 preferred_element_type=jnp.float32)
    s3 = s_full.reshape(_SROWS, _SCOLS, _SCOLS)
    eye = (jax.lax.broadcasted_iota(jnp.int32, (_SROWS, _SCOLS, _SCOLS), 1)
           == jax.lax.broadcasted_iota(jnp.int32, (_SROWS, _SCOLS, _SCOLS), 2))
    scores = jnp.sum(jnp.where(eye, s3, 0.0), axis=1)

    lin = (jax.lax.broadcasted_iota(jnp.int32, (_SROWS, _SCOLS), 0) * _SCOLS
           + jax.lax.broadcasted_iota(jnp.int32, (_SROWS, _SCOLS), 1))

    copies = []
    s = scores
    for j in range(FILT):
        m = jnp.max(s)
        idx = jnp.min(jnp.where(s >= m, lin, jnp.int32(LK)))
        s = jnp.where(lin == idx, -jnp.inf, s)
        fk_ref[pl.ds(j, 1), :] = k_ref[0, pl.ds(idx, 1), :]
        cp = pltpu.make_async_copy(v_ref.at[b, pl.ds(idx, 1), :],
                                   fv_ref.at[pl.ds(j, 1), :], sem)
        cp.start()
        copies.append(cp)

    qb = q_ref[0]
    kf = fk_ref[0:FILT, :]
    probs = []
    for h in range(N_HEAD):
        sl = slice(h * D_H, (h + 1) * D_H)
        qh = qb[:, sl].astype(jnp.bfloat16)
        kh = kf[:, sl].astype(jnp.bfloat16)
        att = jax.lax.dot_general(qh, kh, (((1,), (1,)), ((), ())),
                                  preferred_element_type=jnp.float32)
        att = att - jnp.max(att, axis=1, keepdims=True)
        e = jnp.exp(att)
        probs.append((e / jnp.sum(e, axis=1, keepdims=True)).astype(jnp.bfloat16))

    for cp in copies:
        cp.wait()
    vf = fv_ref[0:FILT, :]
    for h in range(N_HEAD):
        sl = slice(h * D_H, (h + 1) * D_H)
        vh = vf[:, sl].astype(jnp.bfloat16)
        out_ref[0, :, sl] = jax.lax.dot_general(
            probs[h], vh, (((1,), (0,)), ((), ())),
            preferred_element_type=jnp.float32)


def kernel(q, k, v):
    q0t = jnp.swapaxes(q[:, 0:1, :], 1, 2)

    out = pl.pallas_call(
        _fused_kernel,
        grid=(B,),
        in_specs=[
            pl.BlockSpec((1, D, 1), lambda b: (b, 0, 0)),
            pl.BlockSpec((1, LK, D), lambda b: (b, 0, 0)),
            pl.BlockSpec((1, LQ, D), lambda b: (b, 0, 0)),
            pl.BlockSpec(memory_space=pl.ANY),
        ],
        out_specs=pl.BlockSpec((1, LQ, D), lambda b: (b, 0, 0)),
        out_shape=jax.ShapeDtypeStruct((B, LQ, D), jnp.float32),
        scratch_shapes=[
            pltpu.VMEM((16, D), jnp.float32),
            pltpu.VMEM((16, D), jnp.float32),
            pltpu.SemaphoreType.DMA,
        ],
    )(q0t, k, q, v)

    return out


# speedup vs baseline: 3.0326x; 1.0672x over previous
import jax
import jax.numpy as jnp
from jax.experimental import pallas as pl
from jax.experimental.pallas import tpu as pltpu

B = 32
LQ = 32
LK = 2048
D = 1024
N_HEAD = 16
D_H = 64
FILT = 10

_SROWS = 16
_SCOLS = LK // _SROWS


def _fused_kernel(q0_ref, k_ref, q_ref, v_ref, out_ref, fk_ref, fv_ref, sem):
    b = pl.program_id(0)

    kb_bf = k_ref[0].astype(jnp.bfloat16)
    q0_bf = q0_ref[0].astype(jnp.bfloat16)
    scores = jax.lax.dot_general(q0_bf, kb_bf, (((1,), (1,)), ((), ())),
                                 preferred_element_type=jnp.float32)

    lin = jax.lax.broadcasted_iota(jnp.int32, (1, LK), 1)

    copies = []
    s = scores
    for j in range(FILT):
        m = jnp.max(s)
        idx = jnp.min(jnp.where(s >= m, lin, jnp.int32(LK)))
        s = jnp.where(lin == idx, -jnp.inf, s)
        fk_ref[pl.ds(j, 1), :] = k_ref[0, pl.ds(idx, 1), :]
        cp = pltpu.make_async_copy(v_ref.at[b, pl.ds(idx, 1), :],
                                   fv_ref.at[pl.ds(j, 1), :], sem)
        cp.start()
        copies.append(cp)

    qb = q_ref[0]
    kf = fk_ref[0:FILT, :]
    probs = []
    for h in range(N_HEAD):
        sl = slice(h * D_H, (h + 1) * D_H)
        qh = qb[:, sl].astype(jnp.bfloat16)
        kh = kf[:, sl].astype(jnp.bfloat16)
        att = jax.lax.dot_general(qh, kh, (((1,), (1,)), ((), ())),
                                  preferred_element_type=jnp.float32)
        att = att - jnp.max(att, axis=1, keepdims=True)
        e = jnp.exp(att)
        probs.append((e / jnp.sum(e, axis=1, keepdims=True)).astype(jnp.bfloat16))

    for cp in copies:
        cp.wait()
    vf = fv_ref[0:FILT, :]
    for h in range(N_HEAD):
        sl = slice(h * D_H, (h + 1) * D_H)
        vh = vf[:, sl].astype(jnp.bfloat16)
        out_ref[0, :, sl] = jax.lax.dot_general(
            probs[h], vh, (((1,), (0,)), ((), ())),
            preferred_element_type=jnp.float32)


def kernel(q, k, v):
    q0t = q[:, 0:1, :]

    out = pl.pallas_call(
        _fused_kernel,
        grid=(B,),
        in_specs=[
            pl.BlockSpec((1, 1, D), lambda b: (b, 0, 0)),
            pl.BlockSpec((1, LK, D), lambda b: (b, 0, 0)),
            pl.BlockSpec((1, LQ, D), lambda b: (b, 0, 0)),
            pl.BlockSpec(memory_space=pl.ANY),
        ],
        out_specs=pl.BlockSpec((1, LQ, D), lambda b: (b, 0, 0)),
        out_shape=jax.ShapeDtypeStruct((B, LQ, D), jnp.float32),
        scratch_shapes=[
            pltpu.VMEM((16, D), jnp.float32),
            pltpu.VMEM((16, D), jnp.float32),
            pltpu.SemaphoreType.DMA,
        ],
    )(q0t, k, q, v)

    return out


# speedup vs baseline: 3.0382x; 1.0019x over previous
import jax
import jax.numpy as jnp
from jax.experimental import pallas as pl
from jax.experimental.pallas import tpu as pltpu

B = 32
LQ = 32
LK = 2048
D = 1024
N_HEAD = 16
D_H = 64
FILT = 10


def _fused_kernel(q0_ref, k_ref, q_ref, khbm_ref, vhbm_ref, out_ref,
                  scores_ref, fk_ref, fv_ref, sems):
    g = pl.program_id(0)

    prev_b = jnp.clip(g - 1, 0, B - 1)
    slot = jax.lax.rem(g + 1, 2)

    @pl.when(g <= B)
    def _topk():
        s = scores_ref[...]
        lin = jax.lax.broadcasted_iota(jnp.int32, (1, LK), 1)
        for j in range(FILT):
            m = jnp.max(s)
            idx = jnp.min(jnp.where(s >= m, lin, jnp.int32(LK)))
            idx = jnp.minimum(idx, LK - 1)
            s = jnp.where(lin == idx, -jnp.inf, s)
            pltpu.make_async_copy(khbm_ref.at[prev_b, pl.ds(idx, 1), :],
                                  fk_ref.at[slot, pl.ds(j, 1), :],
                                  sems.at[slot]).start()
            pltpu.make_async_copy(vhbm_ref.at[prev_b, pl.ds(idx, 1), :],
                                  fv_ref.at[slot, pl.ds(j, 1), :],
                                  sems.at[slot]).start()

    @pl.when(g < B)
    def _scores():
        q0_bf = q0_ref[0].astype(jnp.bfloat16)
        kb_bf = k_ref[0].astype(jnp.bfloat16)
        scores_ref[...] = jax.lax.dot_general(
            q0_bf, kb_bf, (((1,), (1,)), ((), ())),
            preferred_element_type=jnp.float32)

    aslot = jax.lax.rem(g, 2)

    @pl.when(g >= 1)
    def _drain():
        for j in range(2 * FILT):
            pltpu.make_async_copy(khbm_ref.at[0, pl.ds(0, 1), :],
                                  fk_ref.at[aslot, pl.ds(0, 1), :],
                                  sems.at[aslot]).wait()

    @pl.when(g >= 2)
    def _attn():
        qb = q_ref[0]
        kf = fk_ref[aslot, 0:FILT, :]
        vf = fv_ref[aslot, 0:FILT, :]
        for h in range(N_HEAD):
            sl = slice(h * D_H, (h + 1) * D_H)
            qh = qb[:, sl].astype(jnp.bfloat16)
            kh = kf[:, sl].astype(jnp.bfloat16)
            vh = vf[:, sl].astype(jnp.bfloat16)
            att = jax.lax.dot_general(qh, kh, (((1,), (1,)), ((), ())),
                                      preferred_element_type=jnp.float32)
            att = att - jnp.max(att, axis=1, keepdims=True)
            e = jnp.exp(att)
            p = (e / jnp.sum(e, axis=1, keepdims=True)).astype(jnp.bfloat16)
            out_ref[0, :, sl] = jax.lax.dot_general(
                p, vh, (((1,), (0,)), ((), ())),
                preferred_element_type=jnp.float32)


def kernel(q, k, v):
    q0t = q[:, 0:1, :]

    out = pl.pallas_call(
        _fused_kernel,
        grid=(B + 2,),
        in_specs=[
            pl.BlockSpec((1, 1, D), lambda g: (jnp.minimum(g, B - 1), 0, 0)),
            pl.BlockSpec((1, LK, D), lambda g: (jnp.minimum(g, B - 1), 0, 0)),
            pl.BlockSpec((1, LQ, D), lambda g: (jnp.maximum(g - 2, 0), 0, 0)),
            pl.BlockSpec(memory_space=pl.ANY),
            pl.BlockSpec(memory_space=pl.ANY),
        ],
        out_specs=pl.BlockSpec((1, LQ, D), lambda g: (jnp.maximum(g - 2, 0),
                                                      0, 0)),
        out_shape=jax.ShapeDtypeStruct((B, LQ, D), jnp.float32),
        scratch_shapes=[
            pltpu.VMEM((1, LK), jnp.float32),
            pltpu.VMEM((2, 16, D), jnp.float32),
            pltpu.VMEM((2, 16, D), jnp.float32),
            pltpu.SemaphoreType.DMA((2,)),
        ],
    )(q0t, k, q, k, v)

    return out


# speedup vs baseline: 3.8658x; 1.2724x over previous
import jax
import jax.numpy as jnp
from jax.experimental import pallas as pl
from jax.experimental.pallas import tpu as pltpu

B = 32
LQ = 32
LK = 2048
D = 1024
N_HEAD = 16
D_H = 64
FILT = 10
KSPLIT = 8
KC = D // KSPLIT


def _fused_kernel(q0_ref, k_ref, q_ref, khbm_ref, vhbm_ref, out_ref,
                  scores_ref, fk_ref, fv_ref, sems):
    g = pl.program_id(0)

    prev_b = jnp.clip(g - 1, 0, B - 1)
    slot = jax.lax.rem(g + 1, 2)

    s = scores_ref[...]
    lin = jax.lax.broadcasted_iota(jnp.int32, (1, LK), 1)
    for j in range(FILT):
        m = jnp.max(s)
        idx = jnp.min(jnp.where(s >= m, lin, jnp.int32(LK)))
        idx = jnp.minimum(idx, LK - 1)
        s = jnp.where(lin == idx, -jnp.inf, s)
        pltpu.make_async_copy(khbm_ref.at[prev_b, pl.ds(idx, 1), :],
                              fk_ref.at[slot, pl.ds(j, 1), :],
                              sems.at[slot]).start()
        pltpu.make_async_copy(vhbm_ref.at[prev_b, pl.ds(idx, 1), :],
                              fv_ref.at[slot, pl.ds(j, 1), :],
                              sems.at[slot]).start()

    q0_bf = q0_ref[0].astype(jnp.bfloat16)
    kb_bf = k_ref[0].astype(jnp.bfloat16)
    parts = []
    for c in range(KSPLIT):
        ksl = slice(c * KC, (c + 1) * KC)
        parts.append(jax.lax.dot_general(
            q0_bf[:, ksl], kb_bf[:, ksl], (((1,), (1,)), ((), ())),
            preferred_element_type=jnp.float32))
    snew = parts[0]
    for c in range(1, KSPLIT):
        snew = snew + parts[c]
    scores_ref[...] = snew

    aslot = jax.lax.rem(g, 2)

    @pl.when(g >= 1)
    def _drain():
        for j in range(2 * FILT):
            pltpu.make_async_copy(khbm_ref.at[0, pl.ds(0, 1), :],
                                  fk_ref.at[aslot, pl.ds(0, 1), :],
                                  sems.at[aslot]).wait()

    qb = q_ref[0]
    kf = fk_ref[aslot, 0:FILT, :]
    vf = fv_ref[aslot, 0:FILT, :]
    for h in range(N_HEAD):
        sl = slice(h * D_H, (h + 1) * D_H)
        qh = qb[:, sl].astype(jnp.bfloat16)
        kh = kf[:, sl].astype(jnp.bfloat16)
        vh = vf[:, sl].astype(jnp.bfloat16)
        att = jax.lax.dot_general(qh, kh, (((1,), (1,)), ((), ())),
                                  preferred_element_type=jnp.float32)
        att = att - jnp.max(att, axis=1, keepdims=True)
        e = jnp.exp(att)
        p = (e / jnp.sum(e, axis=1, keepdims=True)).astype(jnp.bfloat16)
        out_ref[0, :, sl] = jax.lax.dot_general(
            p, vh, (((1,), (0,)), ((), ())),
            preferred_element_type=jnp.float32)

    @pl.when(g == B + 1)
    def _final_drain():
        for j in range(2 * FILT):
            pltpu.make_async_copy(khbm_ref.at[0, pl.ds(0, 1), :],
                                  fk_ref.at[slot, pl.ds(0, 1), :],
                                  sems.at[slot]).wait()


def kernel(q, k, v):
    q0t = q[:, 0:1, :]

    out = pl.pallas_call(
        _fused_kernel,
        grid=(B + 2,),
        in_specs=[
            pl.BlockSpec((1, 1, D), lambda g: (jnp.minimum(g, B - 1), 0, 0)),
            pl.BlockSpec((1, LK, D), lambda g: (jnp.minimum(g, B - 1), 0, 0)),
            pl.BlockSpec((1, LQ, D), lambda g: (jnp.maximum(g - 2, 0), 0, 0)),
            pl.BlockSpec(memory_space=pl.ANY),
            pl.BlockSpec(memory_space=pl.ANY),
        ],
        out_specs=pl.BlockSpec((1, LQ, D), lambda g: (jnp.maximum(g - 2, 0),
                                                      0, 0)),
        out_shape=jax.ShapeDtypeStruct((B, LQ, D), jnp.float32),
        scratch_shapes=[
            pltpu.VMEM((1, LK), jnp.float32),
            pltpu.VMEM((2, 16, D), jnp.float32),
            pltpu.VMEM((2, 16, D), jnp.float32),
            pltpu.SemaphoreType.DMA((2,)),
        ],
    )(q0t, k, q, k, v)

    return out


# speedup vs baseline: 3.9293x; 1.0164x over previous
import jax
import jax.numpy as jnp
from jax.experimental import pallas as pl
from jax.experimental.pallas import tpu as pltpu

B = 32
LQ = 32
LK = 2048
D = 1024
N_HEAD = 16
D_H = 64
FILT = 10
KSPLIT = 8
KC = D // KSPLIT


def _fused_kernel(q0_ref, k_ref, q_ref, khbm_ref, vhbm_ref, out_ref,
                  scores_ref, fk_ref, fv_ref, sems):
    g = pl.program_id(0)

    @pl.when(g == 0)
    def _precharge():
        for sz, off in ((8, 0), (1, 8), (1, 9)):
            pltpu.make_async_copy(khbm_ref.at[0, pl.ds(off, sz), :],
                                  fk_ref.at[0, pl.ds(off, sz), :],
                                  sems.at[0]).start()
            pltpu.make_async_copy(vhbm_ref.at[0, pl.ds(off, sz), :],
                                  fv_ref.at[0, pl.ds(off, sz), :],
                                  sems.at[0]).start()

    aslot = jax.lax.rem(g, 2)

    for sz, off in ((8, 0), (1, 8), (1, 9)):
        pltpu.make_async_copy(khbm_ref.at[0, pl.ds(off, sz), :],
                              fk_ref.at[aslot, pl.ds(off, sz), :],
                              sems.at[aslot]).wait()
        pltpu.make_async_copy(vhbm_ref.at[0, pl.ds(off, sz), :],
                              fv_ref.at[aslot, pl.ds(off, sz), :],
                              sems.at[aslot]).wait()

    qb = q_ref[0]
    kf = fk_ref[aslot, 0:FILT, :]
    vf = fv_ref[aslot, 0:FILT, :]
    for h in range(N_HEAD):
        sl = slice(h * D_H, (h + 1) * D_H)
        qh = qb[:, sl].astype(jnp.bfloat16)
        kh = kf[:, sl].astype(jnp.bfloat16)
        vh = vf[:, sl].astype(jnp.bfloat16)
        att = jax.lax.dot_general(qh, kh, (((1,), (1,)), ((), ())),
                                  preferred_element_type=jnp.float32)
        att = att - jnp.max(att, axis=1, keepdims=True)
        e = jnp.exp(att)
        p = (e / jnp.sum(e, axis=1, keepdims=True)).astype(jnp.bfloat16)
        out_ref[0, :, sl] = jax.lax.dot_general(
            p, vh, (((1,), (0,)), ((), ())),
            preferred_element_type=jnp.float32)

    prev_b = jnp.clip(g - 1, 0, B - 1)
    slot = jax.lax.rem(g + 1, 2)

    s = scores_ref[...]
    lin = (jax.lax.broadcasted_iota(jnp.int32, (16, 128), 0) * 128
           + jax.lax.broadcasted_iota(jnp.int32, (16, 128), 1))
    for j in range(FILT):
        m = jnp.max(s)
        idx = jnp.min(jnp.where(s >= m, lin, jnp.int32(LK)))
        idx = jnp.minimum(idx, LK - 1)
        s = jnp.where(lin == idx, -jnp.inf, s)
        pltpu.make_async_copy(khbm_ref.at[prev_b, pl.ds(idx, 1), :],
                              fk_ref.at[slot, pl.ds(j, 1), :],
                              sems.at[slot]).start()
        pltpu.make_async_copy(vhbm_ref.at[prev_b, pl.ds(idx, 1), :],
                              fv_ref.at[slot, pl.ds(j, 1), :],
                              sems.at[slot]).start()

    q0_bf = q0_ref[0].astype(jnp.bfloat16)
    kb_bf = k_ref[0].astype(jnp.bfloat16)
    parts = []
    for c in range(KSPLIT):
        ksl = slice(c * KC, (c + 1) * KC)
        parts.append(jax.lax.dot_general(
            q0_bf[:, ksl], kb_bf[:, ksl], (((1,), (1,)), ((), ())),
            preferred_element_type=jnp.float32))
    snew = parts[0]
    for c in range(1, KSPLIT):
        snew = snew + parts[c]
    scores_ref[...] = jnp.concatenate(
        [snew[:, i * 128:(i + 1) * 128] for i in range(16)], axis=0)

    @pl.when(g == B + 1)
    def _final_drain():
        for sz, off in ((8, 0), (1, 8), (1, 9)):
            pltpu.make_async_copy(khbm_ref.at[0, pl.ds(off, sz), :],
                                  fk_ref.at[slot, pl.ds(off, sz), :],
                                  sems.at[slot]).wait()
            pltpu.make_async_copy(vhbm_ref.at[0, pl.ds(off, sz), :],
                                  fv_ref.at[slot, pl.ds(off, sz), :],
                                  sems.at[slot]).wait()


def kernel(q, k, v):
    q0t = q[:, 0:1, :]

    out = pl.pallas_call(
        _fused_kernel,
        grid=(B + 2,),
        in_specs=[
            pl.BlockSpec((1, 1, D), lambda g: (jnp.minimum(g, B - 1), 0, 0)),
            pl.BlockSpec((1, LK, D), lambda g: (jnp.minimum(g, B - 1), 0, 0)),
            pl.BlockSpec((1, LQ, D), lambda g: (jnp.maximum(g - 2, 0), 0, 0)),
            pl.BlockSpec(memory_space=pl.ANY),
            pl.BlockSpec(memory_space=pl.ANY),
        ],
        out_specs=pl.BlockSpec((1, LQ, D), lambda g: (jnp.maximum(g - 2, 0),
                                                      0, 0)),
        out_shape=jax.ShapeDtypeStruct((B, LQ, D), jnp.float32),
        scratch_shapes=[
            pltpu.VMEM((16, 128), jnp.float32),
            pltpu.VMEM((2, 16, D), jnp.float32),
            pltpu.VMEM((2, 16, D), jnp.float32),
            pltpu.SemaphoreType.DMA((2,)),
        ],
    )(q0t, k, q, k, v)

    return out


# speedup vs baseline: 4.4635x; 1.1359x over previous
import jax
import jax.numpy as jnp
from jax.experimental import pallas as pl
from jax.experimental.pallas import tpu as pltpu

B = 32
LQ = 32
LK = 2048
D = 1024
N_HEAD = 16
D_H = 64
FILT = 10
KSPLIT = 8
KC = D // KSPLIT


def _fused_kernel(q0_ref, k_ref, q_ref, khbm_ref, vhbm_ref, out_ref,
                  scores_ref, fk_ref, fv_ref, sems):
    g = pl.program_id(0)

    @pl.when(g == 0)
    def _precharge():
        for pslot in (2, 0):
            for sz, off in ((8, 0), (1, 8), (1, 9)):
                pltpu.make_async_copy(khbm_ref.at[0, pl.ds(off, sz), :],
                                      fk_ref.at[pslot, pl.ds(off, sz), :],
                                      sems.at[pslot]).start()
                pltpu.make_async_copy(vhbm_ref.at[0, pl.ds(off, sz), :],
                                      fv_ref.at[pslot, pl.ds(off, sz), :],
                                      sems.at[pslot]).start()

    aslot = jax.lax.rem(g + 2, 3)

    for sz, off in ((8, 0), (1, 8), (1, 9)):
        pltpu.make_async_copy(khbm_ref.at[0, pl.ds(off, sz), :],
                              fk_ref.at[aslot, pl.ds(off, sz), :],
                              sems.at[aslot]).wait()
        pltpu.make_async_copy(vhbm_ref.at[0, pl.ds(off, sz), :],
                              fv_ref.at[aslot, pl.ds(off, sz), :],
                              sems.at[aslot]).wait()

    qb = q_ref[0]
    kf = fk_ref[aslot, 0:FILT, :]
    vf = fv_ref[aslot, 0:FILT, :]
    for h in range(N_HEAD):
        sl = slice(h * D_H, (h + 1) * D_H)
        qh = qb[:, sl].astype(jnp.bfloat16)
        kh = kf[:, sl].astype(jnp.bfloat16)
        vh = vf[:, sl].astype(jnp.bfloat16)
        att = jax.lax.dot_general(qh, kh, (((1,), (1,)), ((), ())),
                                  preferred_element_type=jnp.float32)
        att = att - jnp.max(att, axis=1, keepdims=True)
        e = jnp.exp(att)
        p = (e / jnp.sum(e, axis=1, keepdims=True)).astype(jnp.bfloat16)
        out_ref[0, :, sl] = jax.lax.dot_general(
            p, vh, (((1,), (0,)), ((), ())),
            preferred_element_type=jnp.float32)

    prev_b = jnp.clip(g - 1, 0, B - 1)
    slot = jax.lax.rem(g + 1, 3)

    s = scores_ref[...]
    lin = (jax.lax.broadcasted_iota(jnp.int32, (16, 128), 0) * 128
           + jax.lax.broadcasted_iota(jnp.int32, (16, 128), 1))
    for j in range(FILT):
        m = jnp.max(s)
        idx = jnp.min(jnp.where(s >= m, lin, jnp.int32(LK)))
        idx = jnp.minimum(idx, LK - 1)
        s = jnp.where(lin == idx, -jnp.inf, s)
        pltpu.make_async_copy(khbm_ref.at[prev_b, pl.ds(idx, 1), :],
                              fk_ref.at[slot, pl.ds(j, 1), :],
                              sems.at[slot]).start()
        pltpu.make_async_copy(vhbm_ref.at[prev_b, pl.ds(idx, 1), :],
                              fv_ref.at[slot, pl.ds(j, 1), :],
                              sems.at[slot]).start()

    q0_bf = q0_ref[0].astype(jnp.bfloat16)
    kb_bf = k_ref[0].astype(jnp.bfloat16)
    parts = []
    for c in range(KSPLIT):
        ksl = slice(c * KC, (c + 1) * KC)
        parts.append(jax.lax.dot_general(
            q0_bf[:, ksl], kb_bf[:, ksl], (((1,), (1,)), ((), ())),
            preferred_element_type=jnp.float32))
    snew = parts[0]
    for c in range(1, KSPLIT):
        snew = snew + parts[c]
    scores_ref[...] = jnp.concatenate(
        [snew[:, i * 128:(i + 1) * 128] for i in range(16)], axis=0)

    @pl.when(g == B + 2)
    def _final_drain():
        for dslot_shift in (0, 1):
            dslot = jax.lax.rem(g + dslot_shift, 3)
            for sz, off in ((8, 0), (1, 8), (1, 9)):
                pltpu.make_async_copy(khbm_ref.at[0, pl.ds(off, sz), :],
                                      fk_ref.at[dslot, pl.ds(off, sz), :],
                                      sems.at[dslot]).wait()
                pltpu.make_async_copy(vhbm_ref.at[0, pl.ds(off, sz), :],
                                      fv_ref.at[dslot, pl.ds(off, sz), :],
                                      sems.at[dslot]).wait()


def kernel(q, k, v):
    q0t = q[:, 0:1, :]

    out = pl.pallas_call(
        _fused_kernel,
        grid=(B + 3,),
        in_specs=[
            pl.BlockSpec((1, 1, D), lambda g: (jnp.minimum(g, B - 1), 0, 0)),
            pl.BlockSpec((1, LK, D), lambda g: (jnp.minimum(g, B - 1), 0, 0)),
            pl.BlockSpec((1, LQ, D), lambda g: (jnp.maximum(g - 3, 0), 0, 0)),
            pl.BlockSpec(memory_space=pl.ANY),
            pl.BlockSpec(memory_space=pl.ANY),
        ],
        out_specs=pl.BlockSpec((1, LQ, D), lambda g: (jnp.maximum(g - 3, 0),
                                                      0, 0)),
        out_shape=jax.ShapeDtypeStruct((B, LQ, D), jnp.float32),
        scratch_shapes=[
            pltpu.VMEM((16, 128), jnp.float32),
            pltpu.VMEM((3, 16, D), jnp.float32),
            pltpu.VMEM((3, 16, D), jnp.float32),
            pltpu.SemaphoreType.DMA((3,)),
        ],
    )(q0t, k, q, k, v)

    return out
